# baseline (speedup 1.0000x reference)
import functools

import jax
import jax.numpy as jnp
from jax import lax
from jax.experimental import pallas as pl
from jax.experimental.pallas import tpu as pltpu
from jax.experimental.pallas import tpu_sc as plsc

_NC = 2
_NS = 16
_NW = _NC * _NS

_CHUNK = 1024


def _idx_body(clip, t_ref, idx_ref):
    t = t_ref[...]
    d = jnp.abs(t[:, :, None] - t[:, None, :])
    idx_ref[...] = jnp.minimum(d, clip)


def _pairwise_idx(time, clip):
    b, h = time.shape
    blk = 512
    assert b % blk == 0
    return pl.pallas_call(
        functools.partial(_idx_body, clip),
        grid=(b // blk,),
        in_specs=[pl.BlockSpec((blk, h), lambda i: (i, 0))],
        out_specs=pl.BlockSpec((blk, h, h), lambda i: (i, 0, 0)),
        out_shape=jax.ShapeDtypeStruct((b, h, h), jnp.int32),
    )(time)


def _gather_body(
    rows_per_w, idx_hbm, table_hbm, out_hbm, idx_v, rows0, rows1, sem0, sem1
):
    wid = lax.axis_index("s") * _NC + lax.axis_index("c")
    base = wid * rows_per_w
    n_chunks = rows_per_w // _CHUNK
    n2 = n_chunks // 2
    pltpu.sync_copy(idx_hbm.at[pl.ds(base, rows_per_w)], idx_v)

    def gather(c, rows, sem):
        return pltpu.async_copy(
            table_hbm.at[idx_v.at[pl.ds(c * _CHUNK, _CHUNK)]], rows, sem
        )

    def drain_and_write(c, rows, sem):
        pltpu.make_async_copy(
            table_hbm.at[idx_v.at[pl.ds(c * _CHUNK, _CHUNK)]], rows, sem
        ).wait()
        pltpu.sync_copy(rows, out_hbm.at[pl.ds(base + c * _CHUNK, _CHUNK)])

    gather(0, rows0, sem0)

    def body(i, carry):
        c0 = 2 * i
        gather(c0 + 1, rows1, sem1)
        drain_and_write(c0, rows0, sem0)

        @pl.when(i + 1 < n2)
        def _():
            gather(c0 + 2, rows0, sem0)

        drain_and_write(c0 + 1, rows1, sem1)
        return carry

    lax.fori_loop(0, n2, body, 0)


def kernel(time, table, max_len):
    b, h = time.shape
    v, d = table.shape
    clip = v - 1
    idx = _pairwise_idx(time, clip)

    n_rows = b * h * h
    assert n_rows % (_NW * _CHUNK) == 0
    rows_per_w = n_rows // _NW

    idx_flat = idx.reshape(n_rows)

    mesh = plsc.VectorSubcoreMesh(core_axis_name="c", subcore_axis_name="s")
    out = pl.kernel(
        functools.partial(_gather_body, rows_per_w),
        out_type=jax.ShapeDtypeStruct((n_rows, d), jnp.float32),
        mesh=mesh,
        scratch_types=[
            pltpu.VMEM((rows_per_w,), jnp.int32),
            pltpu.VMEM((_CHUNK, d), jnp.float32),
            pltpu.VMEM((_CHUNK, d), jnp.float32),
            pltpu.SemaphoreType.DMA,
            pltpu.SemaphoreType.DMA,
        ],
        compiler_params=pltpu.CompilerParams(use_tc_tiling_on_sc=False),
    )(idx_flat, table)
    return out.reshape(b, h, h, d)


# speedup vs baseline: 3.7291x; 3.7291x over previous
import functools

import jax
import jax.numpy as jnp
from jax import lax
from jax.experimental import pallas as pl
from jax.experimental.pallas import tpu as pltpu
from jax.experimental.pallas import tpu_sc as plsc

_NC = 2
_NS = 16
_NW = _NC * _NS
_L = 16

_CHUNK = 512


def _idx_body(clip, t_ref, idx_ref):
    t = t_ref[...]
    d = jnp.abs(t[:, :, None] - t[:, None, :])
    idx_ref[...] = jnp.minimum(d, clip)


def _pairwise_idx(time, clip):
    b, h = time.shape
    blk = 512
    assert b % blk == 0
    return pl.pallas_call(
        functools.partial(_idx_body, clip),
        grid=(b // blk,),
        in_specs=[pl.BlockSpec((blk, h), lambda i: (i, 0))],
        out_specs=pl.BlockSpec((blk, h, h), lambda i: (i, 0, 0)),
        out_shape=jax.ShapeDtypeStruct((b, h, h), jnp.int32),
    )(time)


def _gather_body(
    rows_per_w,
    d,
    idx_hbm,
    table_hbm,
    out_hbm,
    table_v,
    idx0,
    idx1,
    ob0,
    ob1,
    isem0,
    isem1,
    wsem0,
    wsem1,
):
    wid = lax.axis_index("s") * _NC + lax.axis_index("c")
    row0 = wid * rows_per_w
    n_chunks = rows_per_w // _CHUNK
    n2 = n_chunks // 2
    cw = _CHUNK * d
    n_grp = _CHUNK // _L

    pltpu.sync_copy(table_hbm, table_v)

    lane = lax.iota(jnp.int32, _L)
    sbase = lane * d

    def issue_idx(c, buf, sem):
        return pltpu.async_copy(
            idx_hbm.at[pl.ds(row0 + c * _CHUNK, _CHUNK)], buf, sem
        )

    def wait_idx(buf, sem):
        pltpu.make_async_copy(idx_hbm.at[pl.ds(0, _CHUNK)], buf, sem).wait()

    def issue_write(c, buf, sem):
        return pltpu.async_copy(
            buf, out_hbm.at[pl.ds((row0 + c * _CHUNK) * d, cw)], sem
        )

    def wait_write(buf, sem):
        pltpu.make_async_copy(buf, out_hbm.at[pl.ds(0, cw)], sem).wait()

    def compute(idxb, ob):
        def grp(g, carry):
            rows16 = idxb[pl.ds(g * _L, _L)]
            wb = rows16 * d
            obase = sbase + g * (_L * d)
            for o in range(d):
                v = plsc.load_gather(table_v, [wb + o])
                plsc.store_scatter(ob, [obase + o], v)
            return carry

        lax.fori_loop(0, n_grp, grp, 0)

    issue_idx(0, idx0, isem0)
    issue_idx(1, idx1, isem1)

    def body(i, carry):
        c0 = 2 * i

        wait_idx(idx0, isem0)

        @pl.when(i > 0)
        def _():
            wait_write(ob0, wsem0)

        compute(idx0, ob0)
        issue_write(c0, ob0, wsem0)

        @pl.when(c0 + 2 < n_chunks)
        def _():
            issue_idx(c0 + 2, idx0, isem0)

        wait_idx(idx1, isem1)

        @pl.when(i > 0)
        def _():
            wait_write(ob1, wsem1)

        compute(idx1, ob1)
        issue_write(c0 + 1, ob1, wsem1)

        @pl.when(c0 + 3 < n_chunks)
        def _():
            issue_idx(c0 + 3, idx1, isem1)

        return carry

    lax.fori_loop(0, n2, body, 0)
    wait_write(ob0, wsem0)
    wait_write(ob1, wsem1)


def kernel(time, table, max_len):
    b, h = time.shape
    v, d = table.shape
    clip = v - 1
    idx = _pairwise_idx(time, clip)

    n_rows = b * h * h
    assert n_rows % (_NW * 2 * _CHUNK) == 0
    rows_per_w = n_rows // _NW

    idx_flat = idx.reshape(n_rows)
    table_flat = table.reshape(v * d)

    mesh = plsc.VectorSubcoreMesh(core_axis_name="c", subcore_axis_name="s")
    out = pl.kernel(
        functools.partial(_gather_body, rows_per_w, d),
        out_type=jax.ShapeDtypeStruct((n_rows * d,), jnp.float32),
        mesh=mesh,
        scratch_types=[
            pltpu.VMEM((v * d,), jnp.float32),
            pltpu.VMEM((_CHUNK,), jnp.int32),
            pltpu.VMEM((_CHUNK,), jnp.int32),
            pltpu.VMEM((_CHUNK * d,), jnp.float32),
            pltpu.VMEM((_CHUNK * d,), jnp.float32),
            pltpu.SemaphoreType.DMA,
            pltpu.SemaphoreType.DMA,
            pltpu.SemaphoreType.DMA,
            pltpu.SemaphoreType.DMA,
        ],
        compiler_params=pltpu.CompilerParams(
            use_tc_tiling_on_sc=False, needs_layout_passes=False
        ),
    )(idx_flat, table_flat)
    return out.reshape(b, h, h, d)


# speedup vs baseline: 6.1065x; 1.6375x over previous
import functools

import jax
import jax.numpy as jnp
from jax import lax
from jax.experimental import pallas as pl
from jax.experimental.pallas import tpu as pltpu
from jax.experimental.pallas import tpu_sc as plsc

_NC = 2
_NS = 16
_NW = _NC * _NS
_L = 16


def _body(
    els_per_w,
    h,
    d,
    clip,
    time_hbm,
    table_hbm,
    out_hbm,
    table_v,
    t_v,
    ob0,
    ob1,
    wsem0,
    wsem1,
):
    wid = lax.axis_index("s") * _NC + lax.axis_index("c")
    rows_per_el = h * h
    cw = rows_per_el * d
    n_grp = rows_per_el // _L
    n2 = els_per_w // 2
    word0 = wid * els_per_w * cw

    pltpu.sync_copy(table_hbm, table_v)
    pltpu.sync_copy(time_hbm.at[pl.ds(wid * els_per_w * h, els_per_w * h)], t_v)

    lane = lax.iota(jnp.int32, _L)

    def compute(e, ob):
        ebase = e * h

        @plsc.parallel_loop(0, n_grp, unroll=1)
        def grp(g):
            pv = lane + g * _L
            i = pv // h
            j = pv - i * h
            ti = plsc.load_gather(t_v, [i + ebase])
            tj = plsc.load_gather(t_v, [j + ebase])
            rows16 = jnp.minimum(jnp.abs(ti - tj), clip)
            wb = rows16 * d
            obase = pv * d
            for o in range(d):
                v = plsc.load_gather(table_v, [wb + o])
                plsc.store_scatter(ob, [obase + o], v)

    def issue_write(e, ob, sem):
        pltpu.async_copy(ob, out_hbm.at[pl.ds(word0 + e * cw, cw)], sem)

    def wait_write(ob, sem):
        pltpu.make_async_copy(ob, out_hbm.at[pl.ds(0, cw)], sem).wait()

    def body(it, carry):
        e0 = 2 * it

        @pl.when(it > 0)
        def _():
            wait_write(ob0, wsem0)

        compute(e0, ob0)
        issue_write(e0, ob0, wsem0)

        @pl.when(it > 0)
        def _():
            wait_write(ob1, wsem1)

        compute(e0 + 1, ob1)
        issue_write(e0 + 1, ob1, wsem1)
        return carry

    lax.fori_loop(0, n2, body, 0)
    wait_write(ob0, wsem0)
    wait_write(ob1, wsem1)


def kernel(time, table, max_len):
    b, h = time.shape
    v, d = table.shape
    clip = v - 1

    assert b % (2 * _NW) == 0 and (h * h) % _L == 0
    els_per_w = b // _NW
    n_rows = b * h * h

    mesh = plsc.VectorSubcoreMesh(core_axis_name="c", subcore_axis_name="s")
    out = pl.kernel(
        functools.partial(_body, els_per_w, h, d, clip),
        out_type=jax.ShapeDtypeStruct((n_rows * d,), jnp.float32),
        mesh=mesh,
        scratch_types=[
            pltpu.VMEM((v * d,), jnp.float32),
            pltpu.VMEM((els_per_w * h,), jnp.int32),
            pltpu.VMEM((h * h * d,), jnp.float32),
            pltpu.VMEM((h * h * d,), jnp.float32),
            pltpu.SemaphoreType.DMA,
            pltpu.SemaphoreType.DMA,
        ],
        compiler_params=pltpu.CompilerParams(
            use_tc_tiling_on_sc=False, needs_layout_passes=False
        ),
    )(time.reshape(b * h), table.reshape(v * d))
    return out.reshape(b, h, h, d)


# speedup vs baseline: 13.9145x; 2.2786x over previous
import functools

import jax
import jax.numpy as jnp
from jax import lax
from jax.experimental import pallas as pl
from jax.experimental.pallas import tpu as pltpu
from jax.experimental.pallas import tpu_sc as plsc

_NC = 2
_NS = 16
_NW = _NC * _NS
_L = 16
_BT = 128

_P = 5


def _body(
    h,
    d,
    clip,
    time_hbm,
    table_hbm,
    out_hbm,
    table_v,
    t_v,
    ob0,
    ob1,
    wsem0,
    wsem1,
):
    wid = lax.axis_index("s") * _NC + lax.axis_index("c")
    n_pairs = h * h
    n_chunks = n_pairs // _P
    n2 = n_chunks // 2
    n_g = _BT // _L

    pltpu.sync_copy(table_hbm, table_v)
    pltpu.sync_copy(time_hbm.at[pl.ds(wid * _BT * h, _BT * h)], t_v)

    lane = lax.iota(jnp.int32, _L)
    laneh = lane * h

    def compute(chunk, ob):
        p0 = chunk * _P
        for p_loc in range(_P):
            p = p0 + p_loc
            i = p // h
            q = p - i * h

            @plsc.parallel_loop(0, n_g, unroll=1)
            def grp(g):
                gb = g * (_L * h)
                ti = plsc.load_gather(t_v, [laneh + (gb + i)])
                tq = plsc.load_gather(t_v, [laneh + (gb + q)])
                rows16 = jnp.minimum(jnp.abs(ti - tq), clip)
                wb = rows16 * d
                for c in range(d):
                    v = plsc.load_gather(table_v, [wb + c])
                    ob[p_loc, c // 8, pl.ds((c % 8) * _BT + g * _L, _L)] = v

    def issue_write(chunk, ob, sem):
        pltpu.async_copy(
            ob, out_hbm.at[pl.ds(chunk * _P, _P), :, wid, :], sem
        )

    def wait_write(ob, sem):
        pltpu.make_async_copy(
            ob, out_hbm.at[pl.ds(0, _P), :, wid, :], sem
        ).wait()

    def body(it, carry):
        c0 = 2 * it

        @pl.when(it > 0)
        def _():
            wait_write(ob0, wsem0)

        compute(c0, ob0)
        issue_write(c0, ob0, wsem0)

        @pl.when(it > 0)
        def _():
            wait_write(ob1, wsem1)

        compute(c0 + 1, ob1)
        issue_write(c0 + 1, ob1, wsem1)
        return carry

    lax.fori_loop(0, n2, body, 0)
    wait_write(ob0, wsem0)
    wait_write(ob1, wsem1)


def kernel(time, table, max_len):
    b, h = time.shape
    v, d = table.shape
    clip = v - 1

    n_pairs = h * h
    assert b % (_NW * _BT) == 0 or b == _NW * _BT
    assert d % 8 == 0 and n_pairs % (2 * _P) == 0 and _BT % _L == 0
    nbt = b // _BT
    assert nbt == _NW
    nct = d // 8

    mesh = plsc.VectorSubcoreMesh(core_axis_name="c", subcore_axis_name="s")
    out = pl.kernel(
        functools.partial(_body, h, d, clip),
        out_type=jax.ShapeDtypeStruct((n_pairs, nct, nbt, 8 * _BT), jnp.float32),
        mesh=mesh,
        scratch_types=[
            pltpu.VMEM((v * d,), jnp.float32),
            pltpu.VMEM((_BT * h,), jnp.int32),
            pltpu.VMEM((_P, nct, 8 * _BT), jnp.float32),
            pltpu.VMEM((_P, nct, 8 * _BT), jnp.float32),
            pltpu.SemaphoreType.DMA,
            pltpu.SemaphoreType.DMA,
        ],
        compiler_params=pltpu.CompilerParams(
            use_tc_tiling_on_sc=False, needs_layout_passes=False
        ),
    )(time.reshape(b * h), table.reshape(v * d))
    phys = out.reshape(h, h, nct, nbt, 8, _BT)
    res = phys.transpose(3, 5, 0, 1, 2, 4)
    return res.reshape(b, h, h, d)
